# Initial kernel scaffold; baseline (speedup 1.0000x reference)
import jax
import jax.numpy as jnp
from jax.experimental import pallas as pl


def kernel(x, v, h_appear, block_inds, tables, dW1, db1, dW2, db2, rW1, rb1, rW2, rb2, rW3, rb3):
    raise NotImplementedError("write your pallas kernel here")


# speedup vs baseline: 25.0262x; 25.0262x over previous
import jax
import jax.numpy as jnp
import numpy as np
from jax import lax
from jax.experimental import pallas as pl
from jax.experimental.pallas import tpu as pltpu
from jax.experimental.pallas import tpu_sc as plsc

_N_BLOCKS = 4
_N_LEVELS = 8
_F = 2
_T = 131072
_BASE_RES = 16
_MAX_RES = 2048
_N = 262144
_APPEAR_DIM = 8
_HIDDEN = 64

_b = (_MAX_RES / _BASE_RES) ** (1.0 / (_N_LEVELS - 1))
_RESOLUTIONS = [int(np.floor(_BASE_RES * (_b ** l))) for l in range(_N_LEVELS)]
_P1 = np.uint32(2654435761)
_P2 = np.uint32(805459861)

_NC = 2
_NS = 16
_NW = _NC * _NS
_LANES = 16

_CHUNK = 256
_GRP = _CHUNK // _LANES
_PTS_PER_W = _N // _NW
_NCHUNK = _PTS_PER_W // _CHUNK
_C64 = _CHUNK * _N_LEVELS * 8
_NIDX = 2 * _C64
_IDX_PER_STREAM = 2048
_NSTREAM = _NIDX // _IDX_PER_STREAM

_OFFS = [(i, j, k) for i in (0, 1) for j in (0, 1) for k in (0, 1)]


def _sc_body(xx, xy, xz, bi_hbm, tab_hbm, out_hbm,
             xbuf, ybuf, zbuf, bibuf, fracbuf, idxbuf, rowsf, hbuf, sem):
    wid = lax.axis_index("s") * _NC + lax.axis_index("c")
    iota = lax.iota(jnp.int32, _LANES)

    def chunk_body(ci, carry):
        base = wid * _PTS_PER_W + ci * _CHUNK
        pltpu.sync_copy(xx.at[pl.ds(base, _CHUNK)], xbuf)
        pltpu.sync_copy(xy.at[pl.ds(base, _CHUNK)], ybuf)
        pltpu.sync_copy(xz.at[pl.ds(base, _CHUNK)], zbuf)
        pltpu.sync_copy(bi_hbm.at[pl.ds(base, _CHUNK)], bibuf)

        def grp_a(g, c2):
            off = g * _LANES
            px = xbuf[pl.ds(off, _LANES)]
            py = ybuf[pl.ds(off, _LANES)]
            pz = zbuf[pl.ds(off, _LANES)]
            bi = bibuf[pl.ds(off, _LANES)]
            bbase = bi.astype(jnp.uint32) << jnp.uint32(20)
            for l in range(_N_LEVELS):
                res = jnp.float32(_RESOLUTIONS[l])
                posx = px * res
                posy = py * res
                posz = pz * res
                ix = posx.astype(jnp.int32)
                iy = posy.astype(jnp.int32)
                iz = posz.astype(jnp.int32)
                fracbuf[pl.ds((3 * l + 0) * _CHUNK + off, _LANES)] = posx - ix.astype(jnp.float32)
                fracbuf[pl.ds((3 * l + 1) * _CHUNK + off, _LANES)] = posy - iy.astype(jnp.float32)
                fracbuf[pl.ds((3 * l + 2) * _CHUNK + off, _LANES)] = posz - iz.astype(jnp.float32)
                hx0 = ix.astype(jnp.uint32)
                hx1 = hx0 + jnp.uint32(1)
                hy0 = iy.astype(jnp.uint32) * _P1
                hy1 = hy0 + _P1
                hz0 = iz.astype(jnp.uint32) * _P2
                hz1 = hz0 + _P2
                lb = bbase | jnp.uint32(l << 17)
                for c, (cx, cy, cz) in enumerate(_OFFS):
                    hx = hx1 if cx else hx0
                    hy = hy1 if cy else hy0
                    hz = hz1 if cz else hz0
                    idx = (hx ^ hy ^ hz) & jnp.uint32(_T - 1)
                    e0 = ((lb | idx) << jnp.uint32(1)).astype(jnp.int32)
                    n0 = g * 1024 + l * 128 + c * 16
                    idxbuf[pl.ds(n0, _LANES)] = e0
                    idxbuf[pl.ds(_C64 + n0, _LANES)] = e0 + 1
            return c2

        lax.fori_loop(0, _GRP, grp_a, 0, unroll=False)

        descs = []
        for s in range(_NSTREAM):
            descs.append(pltpu.async_copy(
                tab_hbm.at[idxbuf.at[pl.ds(s * _IDX_PER_STREAM, _IDX_PER_STREAM)]],
                rowsf.at[pl.ds(s * _IDX_PER_STREAM, _IDX_PER_STREAM)],
                sem))
        for d in descs:
            d.wait()

        def grp_b(g, c2):
            off = g * _LANES
            p16 = (off + iota) * jnp.int32(_N_LEVELS * _F)
            for l in range(_N_LEVELS):
                fx = fracbuf[pl.ds((3 * l + 0) * _CHUNK + off, _LANES)]
                fy = fracbuf[pl.ds((3 * l + 1) * _CHUNK + off, _LANES)]
                fz = fracbuf[pl.ds((3 * l + 2) * _CHUNK + off, _LANES)]
                one = jnp.float32(1.0)
                wx = (one - fx, fx)
                wy = (one - fy, fy)
                wz = (one - fz, fz)
                acc0 = jnp.zeros((_LANES,), jnp.float32)
                acc1 = jnp.zeros((_LANES,), jnp.float32)
                for c, (cx, cy, cz) in enumerate(_OFFS):
                    n0 = g * 1024 + l * 128 + c * 16
                    f0 = rowsf[pl.ds(n0, _LANES)]
                    f1 = rowsf[pl.ds(_C64 + n0, _LANES)]
                    w = wx[cx] * wy[cy] * wz[cz]
                    acc0 = acc0 + w * f0
                    acc1 = acc1 + w * f1
                pid = p16 + jnp.int32(2 * l)
                plsc.store_scatter(hbuf, [pid], acc0)
                plsc.store_scatter(hbuf, [pid + 1], acc1)
            return c2

        lax.fori_loop(0, _GRP, grp_b, 0, unroll=False)

        pltpu.sync_copy(hbuf, out_hbm.at[pl.ds(base * (_N_LEVELS * _F), _CHUNK * _N_LEVELS * _F)])
        return carry

    lax.fori_loop(0, _NCHUNK, chunk_body, 0, unroll=False)


def _encode(xx, xy, xz, block_inds, tabf):
    mesh = plsc.VectorSubcoreMesh(core_axis_name="c", subcore_axis_name="s",
                                  num_cores=_NC, num_subcores=_NS)
    f = pl.kernel(
        _sc_body,
        out_type=jax.ShapeDtypeStruct((_N * _N_LEVELS * _F,), jnp.float32),
        mesh=mesh,
        compiler_params=pltpu.CompilerParams(needs_layout_passes=False),
        scratch_types=[
            pltpu.VMEM((_CHUNK,), jnp.float32),
            pltpu.VMEM((_CHUNK,), jnp.float32),
            pltpu.VMEM((_CHUNK,), jnp.float32),
            pltpu.VMEM((_CHUNK,), jnp.int32),
            pltpu.VMEM((3 * _N_LEVELS * _CHUNK,), jnp.float32),
            pltpu.VMEM((_NIDX,), jnp.int32),
            pltpu.VMEM((_NIDX,), jnp.float32),
            pltpu.VMEM((_CHUNK * _N_LEVELS * _F,), jnp.float32),
            pltpu.SemaphoreType.DMA,
        ],
    )
    return f(xx, xy, xz, block_inds, tabf)


def _mlp_body(h_ref, x_ref, v_ref, ha_ref,
              dW1_ref, db1_ref, dW2_ref, db2_ref,
              rW1x_ref, rW1v_ref, rW1h_ref, rW1a_ref, rb1_ref,
              rW2_ref, rb2_ref, rW3_ref, rb3_ref,
              sig_ref, rgb_ref):
    h = h_ref[...]
    d1 = jnp.maximum(jnp.dot(h, dW1_ref[...], preferred_element_type=jnp.float32)
                     + db1_ref[...], 0.0)
    sig_ref[...] = jnp.dot(d1, dW2_ref[...], preferred_element_type=jnp.float32) + db2_ref[...]
    r1 = (jnp.dot(x_ref[...], rW1x_ref[...], preferred_element_type=jnp.float32)
          + jnp.dot(v_ref[...], rW1v_ref[...], preferred_element_type=jnp.float32)
          + jnp.dot(h, rW1h_ref[...], preferred_element_type=jnp.float32)
          + jnp.dot(ha_ref[...], rW1a_ref[...], preferred_element_type=jnp.float32)
          + rb1_ref[...])
    r1 = jnp.maximum(r1, 0.0)
    r2 = jnp.maximum(jnp.dot(r1, rW2_ref[...], preferred_element_type=jnp.float32)
                     + rb2_ref[...], 0.0)
    rgb_ref[...] = jax.nn.sigmoid(
        jnp.dot(r2, rW3_ref[...], preferred_element_type=jnp.float32) + rb3_ref[...])


_MLP_NB = 2048


def _mlp_call(h, x, v, ha, dW1, db1, dW2, db2, rW1, rb1, rW2, rb2, rW3, rb3):
    grid = (_N // _MLP_NB,)
    rW1x, rW1v, rW1h, rW1a = rW1[0:3], rW1[3:6], rW1[6:22], rW1[22:30]

    def rowblk(d):
        return pl.BlockSpec((_MLP_NB, d), lambda i: (i, 0))

    def rep(shape):
        return pl.BlockSpec(shape, lambda i: tuple(0 for _ in shape))

    return pl.pallas_call(
        _mlp_body,
        grid=grid,
        in_specs=[
            rowblk(16), rowblk(3), rowblk(3), rowblk(8),
            rep((16, 64)), rep((1, 64)), rep((64, 1)), rep((1, 1)),
            rep((3, 64)), rep((3, 64)), rep((16, 64)), rep((8, 64)), rep((1, 64)),
            rep((64, 64)), rep((1, 64)), rep((64, 3)), rep((1, 3)),
        ],
        out_specs=[rowblk(1), rowblk(3)],
        out_shape=[
            jax.ShapeDtypeStruct((_N, 1), jnp.float32),
            jax.ShapeDtypeStruct((_N, 3), jnp.float32),
        ],
    )(h, x, v, ha, dW1, db1.reshape(1, 64), dW2, db2.reshape(1, 1),
      rW1x, rW1v, rW1h, rW1a, rb1.reshape(1, 64),
      rW2, rb2.reshape(1, 64), rW3, rb3.reshape(1, 3))


def kernel(x, v, h_appear, block_inds, tables,
           dW1, db1, dW2, db2, rW1, rb1, rW2, rb2, rW3, rb3):
    tabf = tables.reshape(_N_BLOCKS * _N_LEVELS * _T * _F)
    henc = _encode(x[:, 0], x[:, 1], x[:, 2], block_inds, tabf)
    h = henc.reshape(_N, _N_LEVELS * _F)
    sig, rgb = _mlp_call(h, x, v, h_appear,
                         dW1, db1, dW2, db2, rW1, rb1, rW2, rb2, rW3, rb3)
    return (sig[:, 0], rgb)


# speedup vs baseline: 89.3144x; 3.5688x over previous
import jax
import jax.numpy as jnp
import numpy as np
from jax import lax
from jax.experimental import pallas as pl
from jax.experimental.pallas import tpu as pltpu
from jax.experimental.pallas import tpu_sc as plsc

_N_BLOCKS = 4
_N_LEVELS = 8
_F = 2
_T = 131072
_BASE_RES = 16
_MAX_RES = 2048
_N = 262144
_APPEAR_DIM = 8
_HIDDEN = 64

_b = (_MAX_RES / _BASE_RES) ** (1.0 / (_N_LEVELS - 1))
_RESOLUTIONS = [int(np.floor(_BASE_RES * (_b ** l))) for l in range(_N_LEVELS)]
_P1 = np.uint32(2654435761)
_P2 = np.uint32(805459861)

_NC = 2
_NS = 16
_NW = _NC * _NS
_LANES = 16

_CHUNK = 256
_GRP = _CHUNK // _LANES
_PTS_PER_W = _N // _NW
_NCHUNK = _PTS_PER_W // _CHUNK
_C64 = _CHUNK * _N_LEVELS * 8
_NIDX = 2 * _C64
_IDX_PER_STREAM = 2048
_NSTREAM = _NIDX // _IDX_PER_STREAM

_OFFS = [(i, j, k) for i in (0, 1) for j in (0, 1) for k in (0, 1)]


def _sc_body(xx, xy, xz, bi_hbm, tab_hbm, out_hbm,
             xbuf, ybuf, zbuf, bibuf, fracbuf, idxbuf, rowsf, hbuf, sem):
    wid = lax.axis_index("s") * _NC + lax.axis_index("c")
    iota = lax.iota(jnp.int32, _LANES)

    def chunk_body(ci, carry):
        base = wid * _PTS_PER_W + ci * _CHUNK
        pltpu.sync_copy(xx.at[pl.ds(base, _CHUNK)], xbuf)
        pltpu.sync_copy(xy.at[pl.ds(base, _CHUNK)], ybuf)
        pltpu.sync_copy(xz.at[pl.ds(base, _CHUNK)], zbuf)
        pltpu.sync_copy(bi_hbm.at[pl.ds(base, _CHUNK)], bibuf)

        def grp_a(g, c2):
            off = g * _LANES
            px = xbuf[pl.ds(off, _LANES)]
            py = ybuf[pl.ds(off, _LANES)]
            pz = zbuf[pl.ds(off, _LANES)]
            bi = bibuf[pl.ds(off, _LANES)]
            bbase = bi.astype(jnp.uint32) << jnp.uint32(20)
            for l in range(_N_LEVELS):
                res = jnp.float32(_RESOLUTIONS[l])
                posx = px * res
                posy = py * res
                posz = pz * res
                ix = posx.astype(jnp.int32)
                iy = posy.astype(jnp.int32)
                iz = posz.astype(jnp.int32)
                fracbuf[pl.ds((3 * l + 0) * _CHUNK + off, _LANES)] = posx - ix.astype(jnp.float32)
                fracbuf[pl.ds((3 * l + 1) * _CHUNK + off, _LANES)] = posy - iy.astype(jnp.float32)
                fracbuf[pl.ds((3 * l + 2) * _CHUNK + off, _LANES)] = posz - iz.astype(jnp.float32)
                hx0 = ix.astype(jnp.uint32)
                hx1 = hx0 + jnp.uint32(1)
                hy0 = iy.astype(jnp.uint32) * _P1
                hy1 = hy0 + _P1
                hz0 = iz.astype(jnp.uint32) * _P2
                hz1 = hz0 + _P2
                lb = bbase | jnp.uint32(l << 17)
                for c, (cx, cy, cz) in enumerate(_OFFS):
                    hx = hx1 if cx else hx0
                    hy = hy1 if cy else hy0
                    hz = hz1 if cz else hz0
                    idx = (hx ^ hy ^ hz) & jnp.uint32(_T - 1)
                    e0 = (lb | idx).astype(jnp.int32)
                    n0 = g * 1024 + l * 128 + c * 16
                    idxbuf[pl.ds(n0, _LANES)] = e0
                    idxbuf[pl.ds(_C64 + n0, _LANES)] = e0 + jnp.int32(_N_BLOCKS * _N_LEVELS * _T)
            return c2

        lax.fori_loop(0, _GRP, grp_a, 0, unroll=False)

        descs = []
        for s in range(_NSTREAM):
            descs.append(pltpu.async_copy(
                tab_hbm.at[idxbuf.at[pl.ds(s * _IDX_PER_STREAM, _IDX_PER_STREAM)]],
                rowsf.at[pl.ds(s * _IDX_PER_STREAM, _IDX_PER_STREAM)],
                sem))
        for d in descs:
            d.wait()

        def grp_b(g, c2):
            off = g * _LANES
            p16 = (off + iota) * jnp.int32(_N_LEVELS * _F)
            for l in range(_N_LEVELS):
                fx = fracbuf[pl.ds((3 * l + 0) * _CHUNK + off, _LANES)]
                fy = fracbuf[pl.ds((3 * l + 1) * _CHUNK + off, _LANES)]
                fz = fracbuf[pl.ds((3 * l + 2) * _CHUNK + off, _LANES)]
                one = jnp.float32(1.0)
                wx = (one - fx, fx)
                wy = (one - fy, fy)
                wz = (one - fz, fz)
                acc0 = jnp.zeros((_LANES,), jnp.float32)
                acc1 = jnp.zeros((_LANES,), jnp.float32)
                for c, (cx, cy, cz) in enumerate(_OFFS):
                    n0 = g * 1024 + l * 128 + c * 16
                    f0 = rowsf[pl.ds(n0, _LANES)]
                    f1 = rowsf[pl.ds(_C64 + n0, _LANES)]
                    w = wx[cx] * wy[cy] * wz[cz]
                    acc0 = acc0 + w * f0
                    acc1 = acc1 + w * f1
                pid = p16 + jnp.int32(2 * l)
                plsc.store_scatter(hbuf, [pid], acc0)
                plsc.store_scatter(hbuf, [pid + 1], acc1)
            return c2

        lax.fori_loop(0, _GRP, grp_b, 0, unroll=False)

        pltpu.sync_copy(hbuf, out_hbm.at[pl.ds(base * (_N_LEVELS * _F), _CHUNK * _N_LEVELS * _F)])
        return carry

    lax.fori_loop(0, _NCHUNK, chunk_body, 0, unroll=False)


def _encode(xx, xy, xz, block_inds, tabf):
    mesh = plsc.VectorSubcoreMesh(core_axis_name="c", subcore_axis_name="s",
                                  num_cores=_NC, num_subcores=_NS)
    f = pl.kernel(
        _sc_body,
        out_type=jax.ShapeDtypeStruct((_N * _N_LEVELS * _F,), jnp.float32),
        mesh=mesh,
        compiler_params=pltpu.CompilerParams(needs_layout_passes=False),
        scratch_types=[
            pltpu.VMEM((_CHUNK,), jnp.float32),
            pltpu.VMEM((_CHUNK,), jnp.float32),
            pltpu.VMEM((_CHUNK,), jnp.float32),
            pltpu.VMEM((_CHUNK,), jnp.int32),
            pltpu.VMEM((3 * _N_LEVELS * _CHUNK,), jnp.float32),
            pltpu.VMEM((_NIDX,), jnp.int32),
            pltpu.VMEM((_NIDX,), jnp.float32),
            pltpu.VMEM((_CHUNK * _N_LEVELS * _F,), jnp.float32),
            pltpu.SemaphoreType.DMA,
        ],
    )
    return f(xx, xy, xz, block_inds, tabf)


def _mlp_body(h_ref, x_ref, v_ref, ha_ref,
              dW1_ref, db1_ref, dW2_ref, db2_ref,
              rW1x_ref, rW1v_ref, rW1h_ref, rW1a_ref, rb1_ref,
              rW2_ref, rb2_ref, rW3_ref, rb3_ref,
              sig_ref, rgb_ref):
    h = h_ref[...]
    d1 = jnp.maximum(jnp.dot(h, dW1_ref[...], preferred_element_type=jnp.float32)
                     + db1_ref[...], 0.0)
    sig_ref[...] = jnp.dot(d1, dW2_ref[...], preferred_element_type=jnp.float32) + db2_ref[...]
    r1 = (jnp.dot(x_ref[...], rW1x_ref[...], preferred_element_type=jnp.float32)
          + jnp.dot(v_ref[...], rW1v_ref[...], preferred_element_type=jnp.float32)
          + jnp.dot(h, rW1h_ref[...], preferred_element_type=jnp.float32)
          + jnp.dot(ha_ref[...], rW1a_ref[...], preferred_element_type=jnp.float32)
          + rb1_ref[...])
    r1 = jnp.maximum(r1, 0.0)
    r2 = jnp.maximum(jnp.dot(r1, rW2_ref[...], preferred_element_type=jnp.float32)
                     + rb2_ref[...], 0.0)
    rgb_ref[...] = jax.nn.sigmoid(
        jnp.dot(r2, rW3_ref[...], preferred_element_type=jnp.float32) + rb3_ref[...])


_MLP_NB = 2048


def _mlp_call(h, x, v, ha, dW1, db1, dW2, db2, rW1, rb1, rW2, rb2, rW3, rb3):
    grid = (_N // _MLP_NB,)
    rW1x, rW1v, rW1h, rW1a = rW1[0:3], rW1[3:6], rW1[6:22], rW1[22:30]

    def rowblk(d):
        return pl.BlockSpec((_MLP_NB, d), lambda i: (i, 0))

    def rep(shape):
        return pl.BlockSpec(shape, lambda i: tuple(0 for _ in shape))

    return pl.pallas_call(
        _mlp_body,
        grid=grid,
        in_specs=[
            rowblk(16), rowblk(3), rowblk(3), rowblk(8),
            rep((16, 64)), rep((1, 64)), rep((64, 1)), rep((1, 1)),
            rep((3, 64)), rep((3, 64)), rep((16, 64)), rep((8, 64)), rep((1, 64)),
            rep((64, 64)), rep((1, 64)), rep((64, 3)), rep((1, 3)),
        ],
        out_specs=[rowblk(1), rowblk(3)],
        out_shape=[
            jax.ShapeDtypeStruct((_N, 1), jnp.float32),
            jax.ShapeDtypeStruct((_N, 3), jnp.float32),
        ],
    )(h, x, v, ha, dW1, db1.reshape(1, 64), dW2, db2.reshape(1, 1),
      rW1x, rW1v, rW1h, rW1a, rb1.reshape(1, 64),
      rW2, rb2.reshape(1, 64), rW3, rb3.reshape(1, 3))


def kernel(x, v, h_appear, block_inds, tables,
           dW1, db1, dW2, db2, rW1, rb1, rW2, rb2, rW3, rb3):
    tabf = jnp.transpose(tables, (3, 0, 1, 2)).reshape(_F * _N_BLOCKS * _N_LEVELS * _T)
    henc = _encode(x[:, 0], x[:, 1], x[:, 2], block_inds, tabf)
    h = henc.reshape(_N, _N_LEVELS * _F)
    sig, rgb = _mlp_call(h, x, v, h_appear,
                         dW1, db1, dW2, db2, rW1, rb1, rW2, rb2, rW3, rb3)
    return (sig[:, 0], rgb)


# speedup vs baseline: 98.8918x; 1.1072x over previous
import jax
import jax.numpy as jnp
import numpy as np
from jax import lax
from jax.experimental import pallas as pl
from jax.experimental.pallas import tpu as pltpu
from jax.experimental.pallas import tpu_sc as plsc

_N_BLOCKS = 4
_N_LEVELS = 8
_F = 2
_T = 131072
_BASE_RES = 16
_MAX_RES = 2048
_N = 262144
_APPEAR_DIM = 8
_HIDDEN = 64

_b = (_MAX_RES / _BASE_RES) ** (1.0 / (_N_LEVELS - 1))
_RESOLUTIONS = [int(np.floor(_BASE_RES * (_b ** l))) for l in range(_N_LEVELS)]
_P1 = np.uint32(2654435761)
_P2 = np.uint32(805459861)

_NC = 2
_NS = 16
_NW = _NC * _NS
_LANES = 16

_CHUNK = 256
_GRP = _CHUNK // _LANES
_PTS_PER_W = _N // _NW
_NCHUNK = _PTS_PER_W // _CHUNK
_C64 = _CHUNK * _N_LEVELS * 8
_IDX_PER_STREAM = 2048
_NSTREAM = _C64 // _IDX_PER_STREAM
_FRSZ = 3 * _N_LEVELS * _CHUNK
_HSZ = _CHUNK * _N_LEVELS * _F

_OFFS = [(i, j, k) for i in (0, 1) for j in (0, 1) for k in (0, 1)]


def _sc_body(xx, xy, xz, bi_hbm, tab0_hbm, tab1_hbm, out_hbm,
             xbuf, ybuf, zbuf, bibuf, fracbuf, idxbuf, rows0, rows1, hbuf,
             sem0, sem1):
    wid = lax.axis_index("s") * _NC + lax.axis_index("c")
    iota = lax.iota(jnp.int32, _LANES)
    sems = (sem0, sem1)

    def fire(ci, p):
        base = wid * _PTS_PER_W + ci * _CHUNK
        pltpu.sync_copy(xx.at[pl.ds(base, _CHUNK)], xbuf)
        pltpu.sync_copy(xy.at[pl.ds(base, _CHUNK)], ybuf)
        pltpu.sync_copy(xz.at[pl.ds(base, _CHUNK)], zbuf)
        pltpu.sync_copy(bi_hbm.at[pl.ds(base, _CHUNK)], bibuf)
        fof = p * _FRSZ
        iof = p * _C64

        def grp_a(g, c2):
            off = g * _LANES
            px = xbuf[pl.ds(off, _LANES)]
            py = ybuf[pl.ds(off, _LANES)]
            pz = zbuf[pl.ds(off, _LANES)]
            bi = bibuf[pl.ds(off, _LANES)]
            bbase = bi.astype(jnp.uint32) << jnp.uint32(20)
            for l in range(_N_LEVELS):
                res = jnp.float32(_RESOLUTIONS[l])
                posx = px * res
                posy = py * res
                posz = pz * res
                ix = posx.astype(jnp.int32)
                iy = posy.astype(jnp.int32)
                iz = posz.astype(jnp.int32)
                fracbuf[pl.ds(fof + (3 * l + 0) * _CHUNK + off, _LANES)] = posx - ix.astype(jnp.float32)
                fracbuf[pl.ds(fof + (3 * l + 1) * _CHUNK + off, _LANES)] = posy - iy.astype(jnp.float32)
                fracbuf[pl.ds(fof + (3 * l + 2) * _CHUNK + off, _LANES)] = posz - iz.astype(jnp.float32)
                hx0 = ix.astype(jnp.uint32)
                hx1 = hx0 + jnp.uint32(1)
                hy0 = iy.astype(jnp.uint32) * _P1
                hy1 = hy0 + _P1
                hz0 = iz.astype(jnp.uint32) * _P2
                hz1 = hz0 + _P2
                lb = bbase | jnp.uint32(l << 17)
                hyz = (hy0 ^ hz0, hy0 ^ hz1, hy1 ^ hz0, hy1 ^ hz1)
                for c, (cx, cy, cz) in enumerate(_OFFS):
                    hx = hx1 if cx else hx0
                    idx = (hx ^ hyz[cy * 2 + cz]) & jnp.uint32(_T - 1)
                    n0 = g * 1024 + l * 128 + c * 16
                    idxbuf[pl.ds(iof + n0, _LANES)] = (lb | idx).astype(jnp.int32)
            return c2

        lax.fori_loop(0, _GRP, grp_a, 0, unroll=False)

        for s in range(_NSTREAM):
            sl = pl.ds(iof + s * _IDX_PER_STREAM, _IDX_PER_STREAM)
            pltpu.async_copy(tab0_hbm.at[idxbuf.at[sl]], rows0.at[sl], sems[p])
            pltpu.async_copy(tab1_hbm.at[idxbuf.at[sl]], rows1.at[sl], sems[p])

    def finish(ci, p):
        fof = p * _FRSZ
        iof = p * _C64
        for s in range(_NSTREAM):
            sl = pl.ds(iof + s * _IDX_PER_STREAM, _IDX_PER_STREAM)
            src = tab0_hbm.at[pl.ds(0, _IDX_PER_STREAM)]
            pltpu.make_async_copy(src, rows0.at[sl], sems[p]).wait()
            pltpu.make_async_copy(src, rows1.at[sl], sems[p]).wait()
        hof = p * _HSZ

        def grp_b(g, c2):
            off = g * _LANES
            p16 = (off + iota) * jnp.int32(_N_LEVELS * _F)
            for l in range(_N_LEVELS):
                fx = fracbuf[pl.ds(fof + (3 * l + 0) * _CHUNK + off, _LANES)]
                fy = fracbuf[pl.ds(fof + (3 * l + 1) * _CHUNK + off, _LANES)]
                fz = fracbuf[pl.ds(fof + (3 * l + 2) * _CHUNK + off, _LANES)]
                one = jnp.float32(1.0)
                wx = (one - fx, fx)
                wy = (one - fy, fy)
                wz = (one - fz, fz)
                acc0 = jnp.zeros((_LANES,), jnp.float32)
                acc1 = jnp.zeros((_LANES,), jnp.float32)
                for c, (cx, cy, cz) in enumerate(_OFFS):
                    n0 = iof + g * 1024 + l * 128 + c * 16
                    f0 = rows0[pl.ds(n0, _LANES)]
                    f1 = rows1[pl.ds(n0, _LANES)]
                    w = wx[cx] * wy[cy] * wz[cz]
                    acc0 = acc0 + w * f0
                    acc1 = acc1 + w * f1
                pid = hof + p16 + jnp.int32(2 * l)
                plsc.store_scatter(hbuf, [pid], acc0)
                plsc.store_scatter(hbuf, [pid + 1], acc1)
            return c2

        lax.fori_loop(0, _GRP, grp_b, 0, unroll=False)
        base = wid * _PTS_PER_W + ci * _CHUNK
        pltpu.sync_copy(hbuf.at[pl.ds(hof, _HSZ)],
                        out_hbm.at[pl.ds(base * (_N_LEVELS * _F), _HSZ)])

    fire(0, 0)

    def body(io, carry):
        c0 = io * 2
        fire(c0 + 1, 1)
        finish(c0, 0)
        fire(c0 + 2, 0)
        finish(c0 + 1, 1)
        return carry

    lax.fori_loop(0, _NCHUNK // 2 - 1, body, 0, unroll=False)
    fire(_NCHUNK - 1, 1)
    finish(_NCHUNK - 2, 0)
    finish(_NCHUNK - 1, 1)


def _encode(xx, xy, xz, block_inds, tab0, tab1):
    mesh = plsc.VectorSubcoreMesh(core_axis_name="c", subcore_axis_name="s",
                                  num_cores=_NC, num_subcores=_NS)
    f = pl.kernel(
        _sc_body,
        out_type=jax.ShapeDtypeStruct((_N * _N_LEVELS * _F,), jnp.float32),
        mesh=mesh,
        compiler_params=pltpu.CompilerParams(needs_layout_passes=False),
        scratch_types=[
            pltpu.VMEM((_CHUNK,), jnp.float32),
            pltpu.VMEM((_CHUNK,), jnp.float32),
            pltpu.VMEM((_CHUNK,), jnp.float32),
            pltpu.VMEM((_CHUNK,), jnp.int32),
            pltpu.VMEM((2 * _FRSZ,), jnp.float32),
            pltpu.VMEM((2 * _C64,), jnp.int32),
            pltpu.VMEM((2 * _C64,), jnp.float32),
            pltpu.VMEM((2 * _C64,), jnp.float32),
            pltpu.VMEM((2 * _HSZ,), jnp.float32),
            pltpu.SemaphoreType.DMA,
            pltpu.SemaphoreType.DMA,
        ],
    )
    return f(xx, xy, xz, block_inds, tab0, tab1)


def _mlp_body(h_ref, x_ref, v_ref, ha_ref,
              dW1_ref, db1_ref, dW2_ref, db2_ref,
              rW1x_ref, rW1v_ref, rW1h_ref, rW1a_ref, rb1_ref,
              rW2_ref, rb2_ref, rW3_ref, rb3_ref,
              sig_ref, rgb_ref):
    h = h_ref[...]
    d1 = jnp.maximum(jnp.dot(h, dW1_ref[...], preferred_element_type=jnp.float32)
                     + db1_ref[...], 0.0)
    sig_ref[...] = jnp.dot(d1, dW2_ref[...], preferred_element_type=jnp.float32) + db2_ref[...]
    r1 = (jnp.dot(x_ref[...], rW1x_ref[...], preferred_element_type=jnp.float32)
          + jnp.dot(v_ref[...], rW1v_ref[...], preferred_element_type=jnp.float32)
          + jnp.dot(h, rW1h_ref[...], preferred_element_type=jnp.float32)
          + jnp.dot(ha_ref[...], rW1a_ref[...], preferred_element_type=jnp.float32)
          + rb1_ref[...])
    r1 = jnp.maximum(r1, 0.0)
    r2 = jnp.maximum(jnp.dot(r1, rW2_ref[...], preferred_element_type=jnp.float32)
                     + rb2_ref[...], 0.0)
    rgb_ref[...] = jax.nn.sigmoid(
        jnp.dot(r2, rW3_ref[...], preferred_element_type=jnp.float32) + rb3_ref[...])


_MLP_NB = 2048


def _mlp_call(h, x, v, ha, dW1, db1, dW2, db2, rW1, rb1, rW2, rb2, rW3, rb3):
    grid = (_N // _MLP_NB,)
    rW1x, rW1v, rW1h, rW1a = rW1[0:3], rW1[3:6], rW1[6:22], rW1[22:30]

    def rowblk(d):
        return pl.BlockSpec((_MLP_NB, d), lambda i: (i, 0))

    def rep(shape):
        return pl.BlockSpec(shape, lambda i: tuple(0 for _ in shape))

    return pl.pallas_call(
        _mlp_body,
        grid=grid,
        in_specs=[
            rowblk(16), rowblk(3), rowblk(3), rowblk(8),
            rep((16, 64)), rep((1, 64)), rep((64, 1)), rep((1, 1)),
            rep((3, 64)), rep((3, 64)), rep((16, 64)), rep((8, 64)), rep((1, 64)),
            rep((64, 64)), rep((1, 64)), rep((64, 3)), rep((1, 3)),
        ],
        out_specs=[rowblk(1), rowblk(3)],
        out_shape=[
            jax.ShapeDtypeStruct((_N, 1), jnp.float32),
            jax.ShapeDtypeStruct((_N, 3), jnp.float32),
        ],
    )(h, x, v, ha, dW1, db1.reshape(1, 64), dW2, db2.reshape(1, 1),
      rW1x, rW1v, rW1h, rW1a, rb1.reshape(1, 64),
      rW2, rb2.reshape(1, 64), rW3, rb3.reshape(1, 3))


def kernel(x, v, h_appear, block_inds, tables,
           dW1, db1, dW2, db2, rW1, rb1, rW2, rb2, rW3, rb3):
    tp = jnp.transpose(tables, (3, 0, 1, 2))
    tab0 = tp[0].reshape(_N_BLOCKS * _N_LEVELS * _T)
    tab1 = tp[1].reshape(_N_BLOCKS * _N_LEVELS * _T)
    henc = _encode(x[:, 0], x[:, 1], x[:, 2], block_inds, tab0, tab1)
    h = henc.reshape(_N, _N_LEVELS * _F)
    sig, rgb = _mlp_call(h, x, v, h_appear,
                         dW1, db1, dW2, db2, rW1, rb1, rW2, rb2, rW3, rb3)
    return (sig[:, 0], rgb)


# speedup vs baseline: 99.3129x; 1.0043x over previous
import jax
import jax.numpy as jnp
import numpy as np
from jax import lax
from jax.experimental import pallas as pl
from jax.experimental.pallas import tpu as pltpu
from jax.experimental.pallas import tpu_sc as plsc

_N_BLOCKS = 4
_N_LEVELS = 8
_F = 2
_T = 131072
_BASE_RES = 16
_MAX_RES = 2048
_N = 262144
_APPEAR_DIM = 8
_HIDDEN = 64

_b = (_MAX_RES / _BASE_RES) ** (1.0 / (_N_LEVELS - 1))
_RESOLUTIONS = [int(np.floor(_BASE_RES * (_b ** l))) for l in range(_N_LEVELS)]
_P1 = np.uint32(2654435761)
_P2 = np.uint32(805459861)

_NC = 2
_NS = 16
_NW = _NC * _NS
_LANES = 16

_CHUNK = 256
_GRP = _CHUNK // _LANES
_C64 = _CHUNK * _N_LEVELS * 8
_IDX_PER_STREAM = 2048
_NSTREAM = _C64 // _IDX_PER_STREAM
_FRSZ = 3 * _N_LEVELS * _CHUNK
_HSZ = _CHUNK * _N_LEVELS * _F

_OFFS = [(i, j, k) for i in (0, 1) for j in (0, 1) for k in (0, 1)]


def _make_sc_body(pts_per_w, nchunk):
    def _sc_body(xx, xy, xz, bi_hbm, tab0_hbm, tab1_hbm, out_hbm,
                 xbuf, ybuf, zbuf, bibuf, fracbuf, idxbuf, rows0, rows1, hbuf,
                 sem0, sem1):
        return _sc_body_impl(pts_per_w, nchunk,
                             xx, xy, xz, bi_hbm, tab0_hbm, tab1_hbm, out_hbm,
                             xbuf, ybuf, zbuf, bibuf, fracbuf, idxbuf,
                             rows0, rows1, hbuf, sem0, sem1)
    return _sc_body


def _sc_body_impl(_PTS_PER_W, _NCHUNK,
                  xx, xy, xz, bi_hbm, tab0_hbm, tab1_hbm, out_hbm,
                  xbuf, ybuf, zbuf, bibuf, fracbuf, idxbuf, rows0, rows1, hbuf,
                  sem0, sem1):
    wid = lax.axis_index("s") * _NC + lax.axis_index("c")
    iota = lax.iota(jnp.int32, _LANES)
    sems = (sem0, sem1)

    def fire(ci, p):
        base = wid * _PTS_PER_W + ci * _CHUNK
        pltpu.sync_copy(xx.at[pl.ds(base, _CHUNK)], xbuf)
        pltpu.sync_copy(xy.at[pl.ds(base, _CHUNK)], ybuf)
        pltpu.sync_copy(xz.at[pl.ds(base, _CHUNK)], zbuf)
        pltpu.sync_copy(bi_hbm.at[pl.ds(base, _CHUNK)], bibuf)
        fof = p * _FRSZ
        iof = p * _C64

        def grp_a(g, c2):
            off = g * _LANES
            px = xbuf[pl.ds(off, _LANES)]
            py = ybuf[pl.ds(off, _LANES)]
            pz = zbuf[pl.ds(off, _LANES)]
            bi = bibuf[pl.ds(off, _LANES)]
            bbase = bi.astype(jnp.uint32) << jnp.uint32(20)
            for l in range(_N_LEVELS):
                res = jnp.float32(_RESOLUTIONS[l])
                posx = px * res
                posy = py * res
                posz = pz * res
                ix = posx.astype(jnp.int32)
                iy = posy.astype(jnp.int32)
                iz = posz.astype(jnp.int32)
                fracbuf[pl.ds(fof + (3 * l + 0) * _CHUNK + off, _LANES)] = posx - ix.astype(jnp.float32)
                fracbuf[pl.ds(fof + (3 * l + 1) * _CHUNK + off, _LANES)] = posy - iy.astype(jnp.float32)
                fracbuf[pl.ds(fof + (3 * l + 2) * _CHUNK + off, _LANES)] = posz - iz.astype(jnp.float32)
                hx0 = ix.astype(jnp.uint32)
                hx1 = hx0 + jnp.uint32(1)
                hy0 = iy.astype(jnp.uint32) * _P1
                hy1 = hy0 + _P1
                hz0 = iz.astype(jnp.uint32) * _P2
                hz1 = hz0 + _P2
                lb = bbase | jnp.uint32(l << 17)
                hyz = (hy0 ^ hz0, hy0 ^ hz1, hy1 ^ hz0, hy1 ^ hz1)
                for c, (cx, cy, cz) in enumerate(_OFFS):
                    hx = hx1 if cx else hx0
                    idx = (hx ^ hyz[cy * 2 + cz]) & jnp.uint32(_T - 1)
                    n0 = g * 1024 + l * 128 + c * 16
                    idxbuf[pl.ds(iof + n0, _LANES)] = (lb | idx).astype(jnp.int32)
            return c2

        lax.fori_loop(0, _GRP, grp_a, 0, unroll=False)

        for s in range(_NSTREAM):
            sl = pl.ds(iof + s * _IDX_PER_STREAM, _IDX_PER_STREAM)
            pltpu.async_copy(tab0_hbm.at[idxbuf.at[sl]], rows0.at[sl], sems[p])
            pltpu.async_copy(tab1_hbm.at[idxbuf.at[sl]], rows1.at[sl], sems[p])

    def finish(ci, p):
        fof = p * _FRSZ
        iof = p * _C64
        for s in range(_NSTREAM):
            sl = pl.ds(iof + s * _IDX_PER_STREAM, _IDX_PER_STREAM)
            src = tab0_hbm.at[pl.ds(0, _IDX_PER_STREAM)]
            pltpu.make_async_copy(src, rows0.at[sl], sems[p]).wait()
            pltpu.make_async_copy(src, rows1.at[sl], sems[p]).wait()
        hof = p * _HSZ

        def grp_b(g, c2):
            off = g * _LANES
            p16 = (off + iota) * jnp.int32(_N_LEVELS * _F)
            for l in range(_N_LEVELS):
                fx = fracbuf[pl.ds(fof + (3 * l + 0) * _CHUNK + off, _LANES)]
                fy = fracbuf[pl.ds(fof + (3 * l + 1) * _CHUNK + off, _LANES)]
                fz = fracbuf[pl.ds(fof + (3 * l + 2) * _CHUNK + off, _LANES)]
                one = jnp.float32(1.0)
                wx = (one - fx, fx)
                wy = (one - fy, fy)
                wz = (one - fz, fz)
                acc0 = jnp.zeros((_LANES,), jnp.float32)
                acc1 = jnp.zeros((_LANES,), jnp.float32)
                for c, (cx, cy, cz) in enumerate(_OFFS):
                    n0 = iof + g * 1024 + l * 128 + c * 16
                    f0 = rows0[pl.ds(n0, _LANES)]
                    f1 = rows1[pl.ds(n0, _LANES)]
                    w = wx[cx] * wy[cy] * wz[cz]
                    acc0 = acc0 + w * f0
                    acc1 = acc1 + w * f1
                pid = hof + p16 + jnp.int32(2 * l)
                plsc.store_scatter(hbuf, [pid], acc0)
                plsc.store_scatter(hbuf, [pid + 1], acc1)
            return c2

        lax.fori_loop(0, _GRP, grp_b, 0, unroll=False)
        base = wid * _PTS_PER_W + ci * _CHUNK
        pltpu.sync_copy(hbuf.at[pl.ds(hof, _HSZ)],
                        out_hbm.at[pl.ds(base * (_N_LEVELS * _F), _HSZ)])

    fire(0, 0)

    def body(io, carry):
        c0 = io * 2
        fire(c0 + 1, 1)
        finish(c0, 0)
        fire(c0 + 2, 0)
        finish(c0 + 1, 1)
        return carry

    lax.fori_loop(0, _NCHUNK // 2 - 1, body, 0, unroll=False)
    fire(_NCHUNK - 1, 1)
    finish(_NCHUNK - 2, 0)
    finish(_NCHUNK - 1, 1)


def _encode(n, xx, xy, xz, block_inds, tab0, tab1):
    pts_per_w = n // _NW
    nchunk = pts_per_w // _CHUNK
    mesh = plsc.VectorSubcoreMesh(core_axis_name="c", subcore_axis_name="s",
                                  num_cores=_NC, num_subcores=_NS)
    f = pl.kernel(
        _make_sc_body(pts_per_w, nchunk),
        out_type=jax.ShapeDtypeStruct((n * _N_LEVELS * _F,), jnp.float32),
        mesh=mesh,
        compiler_params=pltpu.CompilerParams(needs_layout_passes=False),
        scratch_types=[
            pltpu.VMEM((_CHUNK,), jnp.float32),
            pltpu.VMEM((_CHUNK,), jnp.float32),
            pltpu.VMEM((_CHUNK,), jnp.float32),
            pltpu.VMEM((_CHUNK,), jnp.int32),
            pltpu.VMEM((2 * _FRSZ,), jnp.float32),
            pltpu.VMEM((2 * _C64,), jnp.int32),
            pltpu.VMEM((2 * _C64,), jnp.float32),
            pltpu.VMEM((2 * _C64,), jnp.float32),
            pltpu.VMEM((2 * _HSZ,), jnp.float32),
            pltpu.SemaphoreType.DMA,
            pltpu.SemaphoreType.DMA,
        ],
    )
    return f(xx, xy, xz, block_inds, tab0, tab1)


def _mlp_body(h_ref, x_ref, v_ref, ha_ref,
              dW1_ref, db1_ref, dW2_ref, db2_ref,
              rW1x_ref, rW1v_ref, rW1h_ref, rW1a_ref, rb1_ref,
              rW2_ref, rb2_ref, rW3_ref, rb3_ref,
              sig_ref, rgb_ref):
    h = h_ref[...]
    d1 = jnp.maximum(jnp.dot(h, dW1_ref[...], preferred_element_type=jnp.float32)
                     + db1_ref[...], 0.0)
    sig_ref[...] = jnp.dot(d1, dW2_ref[...], preferred_element_type=jnp.float32) + db2_ref[...]
    r1 = (jnp.dot(x_ref[...], rW1x_ref[...], preferred_element_type=jnp.float32)
          + jnp.dot(v_ref[...], rW1v_ref[...], preferred_element_type=jnp.float32)
          + jnp.dot(h, rW1h_ref[...], preferred_element_type=jnp.float32)
          + jnp.dot(ha_ref[...], rW1a_ref[...], preferred_element_type=jnp.float32)
          + rb1_ref[...])
    r1 = jnp.maximum(r1, 0.0)
    r2 = jnp.maximum(jnp.dot(r1, rW2_ref[...], preferred_element_type=jnp.float32)
                     + rb2_ref[...], 0.0)
    rgb_ref[...] = jax.nn.sigmoid(
        jnp.dot(r2, rW3_ref[...], preferred_element_type=jnp.float32) + rb3_ref[...])


_MLP_NB = 2048


def _mlp_call(n, h, x, v, ha, dW1, db1, dW2, db2, rW1, rb1, rW2, rb2, rW3, rb3):
    grid = (n // _MLP_NB,)
    rW1x, rW1v, rW1h, rW1a = rW1[0:3], rW1[3:6], rW1[6:22], rW1[22:30]

    def rowblk(d):
        return pl.BlockSpec((_MLP_NB, d), lambda i: (i, 0))

    def rep(shape):
        return pl.BlockSpec(shape, lambda i: tuple(0 for _ in shape))

    return pl.pallas_call(
        _mlp_body,
        grid=grid,
        in_specs=[
            rowblk(16), rowblk(3), rowblk(3), rowblk(8),
            rep((16, 64)), rep((1, 64)), rep((64, 1)), rep((1, 1)),
            rep((3, 64)), rep((3, 64)), rep((16, 64)), rep((8, 64)), rep((1, 64)),
            rep((64, 64)), rep((1, 64)), rep((64, 3)), rep((1, 3)),
        ],
        out_specs=[rowblk(1), rowblk(3)],
        out_shape=[
            jax.ShapeDtypeStruct((n, 1), jnp.float32),
            jax.ShapeDtypeStruct((n, 3), jnp.float32),
        ],
    )(h, x, v, ha, dW1, db1.reshape(1, 64), dW2, db2.reshape(1, 1),
      rW1x, rW1v, rW1h, rW1a, rb1.reshape(1, 64),
      rW2, rb2.reshape(1, 64), rW3, rb3.reshape(1, 3))


def kernel(x, v, h_appear, block_inds, tables,
           dW1, db1, dW2, db2, rW1, rb1, rW2, rb2, rW3, rb3):
    tp = jnp.transpose(tables, (3, 0, 1, 2))
    tab0 = tp[0].reshape(_N_BLOCKS * _N_LEVELS * _T)
    tab1 = tp[1].reshape(_N_BLOCKS * _N_LEVELS * _T)
    nh = _N // 2
    sigs, rgbs = [], []
    hs = []
    for i in range(2):
        s = slice(i * nh, (i + 1) * nh)
        henc = _encode(nh, x[s, 0], x[s, 1], x[s, 2], block_inds[s],
                       tab0, tab1)
        hs.append(henc.reshape(nh, _N_LEVELS * _F))
    for i in range(2):
        s = slice(i * nh, (i + 1) * nh)
        sig, rgb = _mlp_call(nh, hs[i], x[s], v[s], h_appear[s],
                             dW1, db1, dW2, db2, rW1, rb1, rW2, rb2, rW3, rb3)
        sigs.append(sig)
        rgbs.append(rgb)
    sig = jnp.concatenate(sigs, axis=0)
    rgb = jnp.concatenate(rgbs, axis=0)
    return (sig[:, 0], rgb)


# speedup vs baseline: 102.2417x; 1.0295x over previous
import jax
import jax.numpy as jnp
import numpy as np
from jax import lax
from jax.experimental import pallas as pl
from jax.experimental.pallas import tpu as pltpu
from jax.experimental.pallas import tpu_sc as plsc

_N_BLOCKS = 4
_N_LEVELS = 8
_F = 2
_T = 131072
_BASE_RES = 16
_MAX_RES = 2048
_N = 262144
_APPEAR_DIM = 8
_HIDDEN = 64

_b = (_MAX_RES / _BASE_RES) ** (1.0 / (_N_LEVELS - 1))
_RESOLUTIONS = [int(np.floor(_BASE_RES * (_b ** l))) for l in range(_N_LEVELS)]
_P1 = np.uint32(2654435761)
_P2 = np.uint32(805459861)

_NC = 2
_NS = 16
_NW = _NC * _NS
_LANES = 16

_CHUNK = 256
_GRP = _CHUNK // _LANES
_C64 = _CHUNK * _N_LEVELS * 8
_IDX_PER_STREAM = 2048
_NSTREAM = _C64 // _IDX_PER_STREAM
_FRSZ = 3 * _N_LEVELS * _CHUNK
_HSZ = _CHUNK * _N_LEVELS * _F

_OFFS = [(i, j, k) for i in (0, 1) for j in (0, 1) for k in (0, 1)]


def _make_sc_body(pts_per_w, nchunk):
    def _sc_body(xx, xy, xz, bi_hbm, tab0_hbm, tab1_hbm, out_hbm,
                 xbuf, ybuf, zbuf, bibuf, fracbuf, idxbuf, rows0, rows1, hbuf,
                 sem0, sem1):
        return _sc_body_impl(pts_per_w, nchunk,
                             xx, xy, xz, bi_hbm, tab0_hbm, tab1_hbm, out_hbm,
                             xbuf, ybuf, zbuf, bibuf, fracbuf, idxbuf,
                             rows0, rows1, hbuf, sem0, sem1)
    return _sc_body


def _sc_body_impl(_PTS_PER_W, _NCHUNK,
                  xx, xy, xz, bi_hbm, tab0_hbm, tab1_hbm, out_hbm,
                  xbuf, ybuf, zbuf, bibuf, fracbuf, idxbuf, rows0, rows1, hbuf,
                  sem0, sem1):
    wid = lax.axis_index("s") * _NC + lax.axis_index("c")
    iota = lax.iota(jnp.int32, _LANES)
    sems = (sem0, sem1)

    def fire(ci, p):
        base = wid * _PTS_PER_W + ci * _CHUNK
        pltpu.sync_copy(xx.at[pl.ds(base, _CHUNK)], xbuf)
        pltpu.sync_copy(xy.at[pl.ds(base, _CHUNK)], ybuf)
        pltpu.sync_copy(xz.at[pl.ds(base, _CHUNK)], zbuf)
        pltpu.sync_copy(bi_hbm.at[pl.ds(base, _CHUNK)], bibuf)
        fof = p * _FRSZ
        iof = p * _C64

        def grp_a(g, c2):
            off = g * _LANES
            px = xbuf[pl.ds(off, _LANES)]
            py = ybuf[pl.ds(off, _LANES)]
            pz = zbuf[pl.ds(off, _LANES)]
            bi = bibuf[pl.ds(off, _LANES)]
            bbase = bi.astype(jnp.uint32) << jnp.uint32(20)
            for l in range(_N_LEVELS):
                res = jnp.float32(_RESOLUTIONS[l])
                posx = px * res
                posy = py * res
                posz = pz * res
                ix = posx.astype(jnp.int32)
                iy = posy.astype(jnp.int32)
                iz = posz.astype(jnp.int32)
                fracbuf[pl.ds(fof + (3 * l + 0) * _CHUNK + off, _LANES)] = posx - ix.astype(jnp.float32)
                fracbuf[pl.ds(fof + (3 * l + 1) * _CHUNK + off, _LANES)] = posy - iy.astype(jnp.float32)
                fracbuf[pl.ds(fof + (3 * l + 2) * _CHUNK + off, _LANES)] = posz - iz.astype(jnp.float32)
                hx0 = ix.astype(jnp.uint32)
                hx1 = hx0 + jnp.uint32(1)
                hy0 = iy.astype(jnp.uint32) * _P1
                hy1 = hy0 + _P1
                hz0 = iz.astype(jnp.uint32) * _P2
                hz1 = hz0 + _P2
                lb = bbase | jnp.uint32(l << 17)
                hyz = (hy0 ^ hz0, hy0 ^ hz1, hy1 ^ hz0, hy1 ^ hz1)
                for c, (cx, cy, cz) in enumerate(_OFFS):
                    hx = hx1 if cx else hx0
                    idx = (hx ^ hyz[cy * 2 + cz]) & jnp.uint32(_T - 1)
                    n0 = g * 1024 + l * 128 + c * 16
                    idxbuf[pl.ds(iof + n0, _LANES)] = (lb | idx).astype(jnp.int32)
            return c2

        lax.fori_loop(0, _GRP, grp_a, 0, unroll=False)

        for s in range(_NSTREAM):
            sl = pl.ds(iof + s * _IDX_PER_STREAM, _IDX_PER_STREAM)
            pltpu.async_copy(tab0_hbm.at[idxbuf.at[sl]], rows0.at[sl], sems[p])
            pltpu.async_copy(tab1_hbm.at[idxbuf.at[sl]], rows1.at[sl], sems[p])

    def finish(ci, p):
        fof = p * _FRSZ
        iof = p * _C64
        for s in range(_NSTREAM):
            sl = pl.ds(iof + s * _IDX_PER_STREAM, _IDX_PER_STREAM)
            src = tab0_hbm.at[pl.ds(0, _IDX_PER_STREAM)]
            pltpu.make_async_copy(src, rows0.at[sl], sems[p]).wait()
            pltpu.make_async_copy(src, rows1.at[sl], sems[p]).wait()
        hof = p * _HSZ

        def grp_b(g, c2):
            off = g * _LANES
            p16 = (off + iota) * jnp.int32(_N_LEVELS * _F)
            for l in range(_N_LEVELS):
                fx = fracbuf[pl.ds(fof + (3 * l + 0) * _CHUNK + off, _LANES)]
                fy = fracbuf[pl.ds(fof + (3 * l + 1) * _CHUNK + off, _LANES)]
                fz = fracbuf[pl.ds(fof + (3 * l + 2) * _CHUNK + off, _LANES)]
                one = jnp.float32(1.0)
                wx = (one - fx, fx)
                wy = (one - fy, fy)
                wz = (one - fz, fz)
                acc0 = jnp.zeros((_LANES,), jnp.float32)
                acc1 = jnp.zeros((_LANES,), jnp.float32)
                for c, (cx, cy, cz) in enumerate(_OFFS):
                    n0 = iof + g * 1024 + l * 128 + c * 16
                    f0 = rows0[pl.ds(n0, _LANES)]
                    f1 = rows1[pl.ds(n0, _LANES)]
                    w = wx[cx] * wy[cy] * wz[cz]
                    acc0 = acc0 + w * f0
                    acc1 = acc1 + w * f1
                pid = hof + p16 + jnp.int32(2 * l)
                plsc.store_scatter(hbuf, [pid], acc0)
                plsc.store_scatter(hbuf, [pid + 1], acc1)
            return c2

        lax.fori_loop(0, _GRP, grp_b, 0, unroll=False)
        base = wid * _PTS_PER_W + ci * _CHUNK
        pltpu.sync_copy(hbuf.at[pl.ds(hof, _HSZ)],
                        out_hbm.at[pl.ds(base * (_N_LEVELS * _F), _HSZ)])

    fire(0, 0)

    def body(io, carry):
        c0 = io * 2
        fire(c0 + 1, 1)
        finish(c0, 0)
        fire(c0 + 2, 0)
        finish(c0 + 1, 1)
        return carry

    lax.fori_loop(0, _NCHUNK // 2 - 1, body, 0, unroll=False)
    fire(_NCHUNK - 1, 1)
    finish(_NCHUNK - 2, 0)
    finish(_NCHUNK - 1, 1)


def _encode(n, xx, xy, xz, block_inds, tab0, tab1):
    pts_per_w = n // _NW
    nchunk = pts_per_w // _CHUNK
    mesh = plsc.VectorSubcoreMesh(core_axis_name="c", subcore_axis_name="s",
                                  num_cores=_NC, num_subcores=_NS)
    f = pl.kernel(
        _make_sc_body(pts_per_w, nchunk),
        out_type=jax.ShapeDtypeStruct((n * _N_LEVELS * _F,), jnp.float32),
        mesh=mesh,
        compiler_params=pltpu.CompilerParams(needs_layout_passes=False),
        scratch_types=[
            pltpu.VMEM((_CHUNK,), jnp.float32),
            pltpu.VMEM((_CHUNK,), jnp.float32),
            pltpu.VMEM((_CHUNK,), jnp.float32),
            pltpu.VMEM((_CHUNK,), jnp.int32),
            pltpu.VMEM((2 * _FRSZ,), jnp.float32),
            pltpu.VMEM((2 * _C64,), jnp.int32),
            pltpu.VMEM((2 * _C64,), jnp.float32),
            pltpu.VMEM((2 * _C64,), jnp.float32),
            pltpu.VMEM((2 * _HSZ,), jnp.float32),
            pltpu.SemaphoreType.DMA,
            pltpu.SemaphoreType.DMA,
        ],
    )
    return f(xx, xy, xz, block_inds, tab0, tab1)


def _mlp_body(h_ref, x_ref, v_ref, ha_ref,
              dW1_ref, db1_ref, dW2_ref, db2_ref,
              rW1x_ref, rW1v_ref, rW1h_ref, rW1a_ref, rb1_ref,
              rW2_ref, rb2_ref, rW3_ref, rb3_ref,
              sig_ref, rgb_ref):
    h = h_ref[...]
    d1 = jnp.maximum(jnp.dot(h, dW1_ref[...], preferred_element_type=jnp.float32)
                     + db1_ref[...], 0.0)
    sig_ref[...] = jnp.dot(d1, dW2_ref[...], preferred_element_type=jnp.float32) + db2_ref[...]
    r1 = (jnp.dot(x_ref[...], rW1x_ref[...], preferred_element_type=jnp.float32)
          + jnp.dot(v_ref[...], rW1v_ref[...], preferred_element_type=jnp.float32)
          + jnp.dot(h, rW1h_ref[...], preferred_element_type=jnp.float32)
          + jnp.dot(ha_ref[...], rW1a_ref[...], preferred_element_type=jnp.float32)
          + rb1_ref[...])
    r1 = jnp.maximum(r1, 0.0)
    r2 = jnp.maximum(jnp.dot(r1, rW2_ref[...], preferred_element_type=jnp.float32)
                     + rb2_ref[...], 0.0)
    rgb_ref[...] = jax.nn.sigmoid(
        jnp.dot(r2, rW3_ref[...], preferred_element_type=jnp.float32) + rb3_ref[...])


_MLP_NB = 2048


def _mlp_call(n, h, x, v, ha, dW1, db1, dW2, db2, rW1, rb1, rW2, rb2, rW3, rb3):
    grid = (n // _MLP_NB,)
    rW1x, rW1v, rW1h, rW1a = rW1[0:3], rW1[3:6], rW1[6:22], rW1[22:30]

    def rowblk(d):
        return pl.BlockSpec((_MLP_NB, d), lambda i: (i, 0))

    def rep(shape):
        return pl.BlockSpec(shape, lambda i: tuple(0 for _ in shape))

    return pl.pallas_call(
        _mlp_body,
        grid=grid,
        in_specs=[
            rowblk(16), rowblk(3), rowblk(3), rowblk(8),
            rep((16, 64)), rep((1, 64)), rep((64, 1)), rep((1, 1)),
            rep((3, 64)), rep((3, 64)), rep((16, 64)), rep((8, 64)), rep((1, 64)),
            rep((64, 64)), rep((1, 64)), rep((64, 3)), rep((1, 3)),
        ],
        out_specs=[rowblk(1), rowblk(3)],
        out_shape=[
            jax.ShapeDtypeStruct((n, 1), jnp.float32),
            jax.ShapeDtypeStruct((n, 3), jnp.float32),
        ],
    )(h, x, v, ha, dW1, db1.reshape(1, 64), dW2, db2.reshape(1, 1),
      rW1x, rW1v, rW1h, rW1a, rb1.reshape(1, 64),
      rW2, rb2.reshape(1, 64), rW3, rb3.reshape(1, 3))


def kernel(x, v, h_appear, block_inds, tables,
           dW1, db1, dW2, db2, rW1, rb1, rW2, rb2, rW3, rb3):
    tp = jnp.transpose(tables, (3, 0, 1, 2))
    tab0 = tp[0].reshape(_N_BLOCKS * _N_LEVELS * _T)
    tab1 = tp[1].reshape(_N_BLOCKS * _N_LEVELS * _T)
    nsplit = 4
    nh = _N // nsplit
    sigs, rgbs = [], []
    hs = []
    for i in range(nsplit):
        s = slice(i * nh, (i + 1) * nh)
        henc = _encode(nh, x[s, 0], x[s, 1], x[s, 2], block_inds[s],
                       tab0, tab1)
        hs.append(henc.reshape(nh, _N_LEVELS * _F))
    for i in range(nsplit):
        s = slice(i * nh, (i + 1) * nh)
        sig, rgb = _mlp_call(nh, hs[i], x[s], v[s], h_appear[s],
                             dW1, db1, dW2, db2, rW1, rb1, rW2, rb2, rW3, rb3)
        sigs.append(sig)
        rgbs.append(rgb)
    sig = jnp.concatenate(sigs, axis=0)
    rgb = jnp.concatenate(rgbs, axis=0)
    return (sig[:, 0], rgb)
